# baseline (speedup 1.0000x reference)
import functools

import jax
import jax.numpy as jnp
from jax import lax
from jax.experimental import pallas as pl
from jax.experimental.pallas import tpu as pltpu
from jax.experimental.pallas import tpu_sc as plsc

ROWS = 6400000
COLS = 5
SEGS = 100000

SB = 400
NBLK = 50000 // SB


def _hcols_body(x_ref, h0, h1, h2, h3, h4):
    xb = x_ref[...]
    m = jnp.max(xb, axis=0)
    sh = xb - m[None]
    ls = jnp.log(jnp.sum(jnp.exp(sh), axis=0))
    outs = (h0, h1, h2, h3, h4)
    for j in range(COLS):
        outs[j][...] = (sh[j] - ls).reshape(SB * 128)


def _h_columns(x):
    xt3 = x.reshape(50000, 128, COLS).transpose(2, 0, 1)
    return pl.pallas_call(
        _hcols_body,
        out_shape=[jax.ShapeDtypeStruct((ROWS,), jnp.float32)] * COLS,
        grid=(NBLK,),
        in_specs=[pl.BlockSpec((COLS, SB, 128), lambda i: (0, i, 0))],
        out_specs=[pl.BlockSpec((SB * 128,), lambda i: (i,))] * COLS,
        compiler_params=pltpu.CompilerParams(
            dimension_semantics=("arbitrary",),
        ),
    )(xt3)


NC = 2
NS = 16
NW = NC * NS
RPW = ROWS // NW
P = 10000
CH = RPW // P
SEG_PAD = 100096
RG = SEG_PAD // NS
NA = COLS + 1


def _segsum_body(h0, h1, h2, h3, h4, batch_hbm, out_hbm,
                 bv0, bv1, vv0, vv1, ones_v, zv,
                 semb, semv0, semv1,
                 sh0, sh1, sh2, sh3, sh4, shc):
    cid = lax.axis_index("c")
    sid = lax.axis_index("s")
    wid = cid * NS + sid
    hs = (h0, h1, h2, h3, h4)
    shs = (sh0, sh1, sh2, sh3, sh4, shc)
    bvs = (bv0, bv1)
    vvs = (vv0, vv1)
    semvs = (semv0, semv1)

    def zfill(i, _):
        zv[pl.ds(i * 16, 16)] = jnp.zeros((16,), jnp.float32)
        return 0
    lax.fori_loop(0, RG // 16, zfill, 0)

    def ofill(i, _):
        ones_v[pl.ds(i * 16, 16)] = jnp.ones((16,), jnp.float32)
        return 0
    lax.fori_loop(0, P // 16, ofill, 0)

    reg = pl.ds(sid * RG, RG)
    for a in range(NA):
        pltpu.sync_copy(zv, shs[a].at[reg])
    plsc.subcore_barrier()

    base = wid * RPW

    def off(c):
        return pl.ds(pl.multiple_of(base + c * P, 16), P)

    pend_b = pltpu.async_copy(batch_hbm.at[off(0)], bv0, semb)
    pend_v = pltpu.async_copy(h0.at[off(0)], vv0, semv0)
    t = 0
    for c in range(CH):
        pend_b.wait()
        bvc = bvs[c % 2]
        if c + 1 < CH:
            pend_b = pltpu.async_copy(batch_hbm.at[off(c + 1)],
                                      bvs[(c + 1) % 2], semb)
        for j in range(COLS):
            pend_v.wait()
            cur = vvs[t % 2]
            nxt = vvs[(t + 1) % 2]
            if j + 1 < COLS:
                pend_v = pltpu.async_copy(hs[j + 1].at[off(c)], nxt,
                                          semvs[(t + 1) % 2])
            elif c + 1 < CH:
                pend_v = pltpu.async_copy(h0.at[off(c + 1)], nxt,
                                          semvs[(t + 1) % 2])
            t += 1
            pltpu.sync_copy(cur, shs[j].at[bvc], add=True)
        pltpu.sync_copy(ones_v, shc.at[bvc], add=True)

    plsc.subcore_barrier()
    for a in range(NA):
        ooff = pl.multiple_of((cid * NA + a) * SEG_PAD + sid * RG, 16)
        pltpu.sync_copy(shs[a].at[reg], zv)
        pltpu.sync_copy(zv, out_hbm.at[pl.ds(ooff, RG)])


def _segment_sums(hcols, batch):
    mesh = plsc.VectorSubcoreMesh(core_axis_name="c", subcore_axis_name="s")
    f = functools.partial(
        pl.kernel,
        out_type=jax.ShapeDtypeStruct((NC * NA * SEG_PAD,), jnp.float32),
        mesh=mesh,
        scratch_types=[
            pltpu.VMEM((P,), jnp.int32),
            pltpu.VMEM((P,), jnp.int32),
            pltpu.VMEM((P,), jnp.float32),
            pltpu.VMEM((P,), jnp.float32),
            pltpu.VMEM((P,), jnp.float32),
            pltpu.VMEM((RG,), jnp.float32),
            pltpu.SemaphoreType.DMA,
            pltpu.SemaphoreType.DMA,
            pltpu.SemaphoreType.DMA,
        ] + [pltpu.VMEM_SHARED((SEG_PAD,), jnp.float32)] * NA,
    )(_segsum_body)
    return f(*hcols, batch)


def _final_body(acc_ref, wb_ref, b_ref, out_ref):
    def region(a):
        lo = acc_ref[pl.ds(a * SEG_PAD, SEG_PAD)]
        hi = acc_ref[pl.ds((NA + a) * SEG_PAD, SEG_PAD)]
        return lo + hi

    def round_bf16(v):
        bits = lax.bitcast_convert_type(v, jnp.int32)
        lsb = lax.shift_right_logical(bits, 16) & 1
        bits = (bits + 0x7FFF + lsb) & jnp.int32(-65536)
        return lax.bitcast_convert_type(bits, jnp.float32)

    n = jnp.maximum(region(COLS), 1.0)
    out = jnp.zeros((SEG_PAD,), jnp.float32) + b_ref[0]
    for j in range(COLS):
        mj = round_bf16(region(j) / n)
        out = out + mj * round_bf16(wb_ref[0, j])
    out_ref[...] = out


def _finalize(acc, wb, b):
    return pl.pallas_call(
        _final_body,
        out_shape=jax.ShapeDtypeStruct((SEG_PAD,), jnp.float32),
        in_specs=[
            pl.BlockSpec((NC * NA * SEG_PAD,), lambda: (0,)),
            pl.BlockSpec(memory_space=pltpu.SMEM),
            pl.BlockSpec(memory_space=pltpu.SMEM),
        ],
        out_specs=pl.BlockSpec((SEG_PAD,), lambda: (0,)),
    )(acc, wb, b)


def kernel(x, batch, W, b):
    hcols = _h_columns(x)
    acc = _segment_sums(hcols, batch.astype(jnp.int32))
    wb = W.astype(jnp.bfloat16).astype(jnp.float32)
    out1 = _finalize(acc, wb, b.astype(jnp.float32))
    return out1[:SEGS].reshape(SEGS, 1)


# speedup vs baseline: 1.0698x; 1.0698x over previous
import functools

import jax
import jax.numpy as jnp
from jax import lax
from jax.experimental import pallas as pl
from jax.experimental.pallas import tpu as pltpu
from jax.experimental.pallas import tpu_sc as plsc

ROWS = 6400000
COLS = 5
SEGS = 100000

SB = 400
NBLK = 50000 // SB


def _hcols_body(x_ref, h0, h1, h2, h3, h4):
    xb = x_ref[...]
    m = jnp.max(xb, axis=0)
    sh = xb - m[None]
    ls = jnp.log(jnp.sum(jnp.exp(sh), axis=0))
    outs = (h0, h1, h2, h3, h4)
    for j in range(COLS):
        outs[j][...] = (sh[j] - ls).reshape(SB * 128)


def _h_columns(x):
    xt3 = x.reshape(50000, 128, COLS).transpose(2, 0, 1)
    return pl.pallas_call(
        _hcols_body,
        out_shape=[jax.ShapeDtypeStruct((ROWS,), jnp.float32)] * COLS,
        grid=(NBLK,),
        in_specs=[pl.BlockSpec((COLS, SB, 128), lambda i: (0, i, 0))],
        out_specs=[pl.BlockSpec((SB * 128,), lambda i: (i,))] * COLS,
        compiler_params=pltpu.CompilerParams(
            dimension_semantics=("arbitrary",),
        ),
    )(xt3)


NC = 2
NS = 16
NW = NC * NS
RPW = ROWS // NW
P = 10000
CH = RPW // P
SEG_PAD = 100096
RG = SEG_PAD // NS
NA = COLS + 1


def _segsum_body(h0, h1, h2, h3, h4, batch_hbm, out_hbm,
                 bv0, bv1, vv0, vv1, zv,
                 semb, semv0, semv1,
                 sh0, sh1, sh2, sh3, sh4):
    cid = lax.axis_index("c")
    sid = lax.axis_index("s")
    wid = cid * NS + sid
    hs = (h0, h1, h2, h3, h4)
    shs = (sh0, sh1, sh2, sh3, sh4)
    bvs = (bv0, bv1)
    vvs = (vv0, vv1)
    semvs = (semv0, semv1)

    def zfill(i, _):
        zv[pl.ds(i * 16, 16)] = jnp.zeros((16,), jnp.float32)
        return 0
    lax.fori_loop(0, RG // 16, zfill, 0)

    reg = pl.ds(sid * RG, RG)
    for a in range(COLS):
        pltpu.sync_copy(zv, shs[a].at[reg])
    plsc.subcore_barrier()

    base = wid * RPW

    def off(c):
        return pl.ds(pl.multiple_of(base + c * P, 16), P)

    pend_b = pltpu.async_copy(batch_hbm.at[off(0)], bv0, semb)
    pend_v = pltpu.async_copy(h0.at[off(0)], vv0, semv0)
    t = 0
    for c in range(CH):
        pend_b.wait()
        bvc = bvs[c % 2]
        if c + 1 < CH:
            pend_b = pltpu.async_copy(batch_hbm.at[off(c + 1)],
                                      bvs[(c + 1) % 2], semb)
        for j in range(COLS):
            pend_v.wait()
            cur = vvs[t % 2]
            nxt = vvs[(t + 1) % 2]
            if j + 1 < COLS:
                pend_v = pltpu.async_copy(hs[j + 1].at[off(c)], nxt,
                                          semvs[(t + 1) % 2])
            elif c + 1 < CH:
                pend_v = pltpu.async_copy(h0.at[off(c + 1)], nxt,
                                          semvs[(t + 1) % 2])
            t += 1
            pltpu.sync_copy(cur, shs[j].at[bvc], add=True)

    plsc.subcore_barrier()
    for a in range(COLS):
        ooff = pl.multiple_of((cid * COLS + a) * SEG_PAD + sid * RG, 16)
        pltpu.sync_copy(shs[a].at[reg], zv)
        pltpu.sync_copy(zv, out_hbm.at[pl.ds(ooff, RG)])


def _segment_sums(hcols, batch):
    mesh = plsc.VectorSubcoreMesh(core_axis_name="c", subcore_axis_name="s")
    f = functools.partial(
        pl.kernel,
        out_type=jax.ShapeDtypeStruct((NC * COLS * SEG_PAD,), jnp.float32),
        mesh=mesh,
        scratch_types=[
            pltpu.VMEM((P,), jnp.int32),
            pltpu.VMEM((P,), jnp.int32),
            pltpu.VMEM((P,), jnp.float32),
            pltpu.VMEM((P,), jnp.float32),
            pltpu.VMEM((RG,), jnp.float32),
            pltpu.SemaphoreType.DMA,
            pltpu.SemaphoreType.DMA,
            pltpu.SemaphoreType.DMA,
        ] + [pltpu.VMEM_SHARED((SEG_PAD,), jnp.float32)] * COLS,
    )(_segsum_body)
    return f(*hcols, batch)


def _counts_body(batch_hbm, out_hbm, bv0, bv1, ones_v, zv, semb, shc):
    cid = lax.axis_index("c")
    sid = lax.axis_index("s")
    wid = cid * NS + sid
    bvs = (bv0, bv1)

    def zfill(i, _):
        zv[pl.ds(i * 16, 16)] = jnp.zeros((16,), jnp.float32)
        return 0
    lax.fori_loop(0, RG // 16, zfill, 0)

    def ofill(i, _):
        ones_v[pl.ds(i * 16, 16)] = jnp.ones((16,), jnp.float32)
        return 0
    lax.fori_loop(0, P // 16, ofill, 0)

    reg = pl.ds(sid * RG, RG)
    pltpu.sync_copy(zv, shc.at[reg])
    plsc.subcore_barrier()

    base = wid * RPW

    def off(c):
        return pl.ds(pl.multiple_of(base + c * P, 16), P)

    pend_b = pltpu.async_copy(batch_hbm.at[off(0)], bv0, semb)
    for c in range(CH):
        pend_b.wait()
        bvc = bvs[c % 2]
        if c + 1 < CH:
            pend_b = pltpu.async_copy(batch_hbm.at[off(c + 1)],
                                      bvs[(c + 1) % 2], semb)
        pltpu.sync_copy(ones_v, shc.at[bvc], add=True)

    plsc.subcore_barrier()
    ooff = pl.multiple_of(cid * SEG_PAD + sid * RG, 16)
    pltpu.sync_copy(shc.at[reg], zv)
    pltpu.sync_copy(zv, out_hbm.at[pl.ds(ooff, RG)])


def _count_sums(batch):
    mesh = plsc.VectorSubcoreMesh(core_axis_name="c", subcore_axis_name="s")
    f = functools.partial(
        pl.kernel,
        out_type=jax.ShapeDtypeStruct((NC * SEG_PAD,), jnp.float32),
        mesh=mesh,
        scratch_types=[
            pltpu.VMEM((P,), jnp.int32),
            pltpu.VMEM((P,), jnp.int32),
            pltpu.VMEM((P,), jnp.float32),
            pltpu.VMEM((RG,), jnp.float32),
            pltpu.SemaphoreType.DMA,
            pltpu.VMEM_SHARED((SEG_PAD,), jnp.float32),
        ],
    )(_counts_body)
    return f(batch)


def _final_body(acc_ref, cacc_ref, wb_ref, b_ref, out_ref):
    def region(a):
        lo = acc_ref[pl.ds(a * SEG_PAD, SEG_PAD)]
        hi = acc_ref[pl.ds((COLS + a) * SEG_PAD, SEG_PAD)]
        return lo + hi

    def cnts():
        return (cacc_ref[pl.ds(0, SEG_PAD)]
                + cacc_ref[pl.ds(SEG_PAD, SEG_PAD)])

    def round_bf16(v):
        bits = lax.bitcast_convert_type(v, jnp.int32)
        lsb = lax.shift_right_logical(bits, 16) & 1
        bits = (bits + 0x7FFF + lsb) & jnp.int32(-65536)
        return lax.bitcast_convert_type(bits, jnp.float32)

    n = jnp.maximum(cnts(), 1.0)
    out = jnp.zeros((SEG_PAD,), jnp.float32) + b_ref[0]
    for j in range(COLS):
        mj = round_bf16(region(j) / n)
        out = out + mj * round_bf16(wb_ref[0, j])
    out_ref[...] = out


def _finalize(acc, cacc, wb, b):
    return pl.pallas_call(
        _final_body,
        out_shape=jax.ShapeDtypeStruct((SEG_PAD,), jnp.float32),
        in_specs=[
            pl.BlockSpec((NC * COLS * SEG_PAD,), lambda: (0,)),
            pl.BlockSpec((NC * SEG_PAD,), lambda: (0,)),
            pl.BlockSpec(memory_space=pltpu.SMEM),
            pl.BlockSpec(memory_space=pltpu.SMEM),
        ],
        out_specs=pl.BlockSpec((SEG_PAD,), lambda: (0,)),
    )(acc, cacc, wb, b)


def kernel(x, batch, W, b):
    bi = batch.astype(jnp.int32)
    cacc = _count_sums(bi)
    hcols = _h_columns(x)
    acc = _segment_sums(hcols, bi)
    out1 = _finalize(acc, cacc, W.astype(jnp.float32),
                     b.astype(jnp.float32))
    return out1[:SEGS].reshape(SEGS, 1)
